# Initial kernel scaffold; baseline (speedup 1.0000x reference)
import jax
import jax.numpy as jnp
from jax.experimental import pallas as pl


def kernel(x, edge_index, edge_weight, W1, b1, W2, b2):
    raise NotImplementedError("write your pallas kernel here")


# speedup vs baseline: 30.4637x; 30.4637x over previous
import functools

import jax
import jax.numpy as jnp
from jax import lax
from jax.experimental import pallas as pl
from jax.experimental.pallas import tpu as pltpu
from jax.experimental.pallas import tpu_sc as plsc

NC, NS, L = 2, 16, 16
NT = NC * NS
CHUNK = 128


def _cdiv(a, b):
    return (a + b - 1) // b


def _deg_body(nrows, kch, col_h, w_h, out_h, col_v, w_v, zb_v, acc_sh):
    c = lax.axis_index("c")
    s = lax.axis_index("s")
    wid = c * NS + s
    for k in range(nrows // L):
        zb_v[pl.ds(k * L, L)] = jnp.zeros((L,), jnp.float32)
    pltpu.sync_copy(zb_v, acc_sh.at[pl.ds(s * nrows, nrows)])
    pltpu.sync_copy(col_h.at[wid], col_v)
    pltpu.sync_copy(w_h.at[wid], w_v)
    plsc.subcore_barrier()

    def body(j, carry):
        pltpu.sync_copy(w_v.at[j], acc_sh.at[col_v.at[j]], add=True)
        return carry

    lax.fori_loop(0, kch, body, 0)
    plsc.subcore_barrier()
    pltpu.sync_copy(acc_sh.at[pl.ds(s * nrows, nrows)],
                    out_h.at[c, pl.ds(s * nrows, nrows)])


def _prop_body(nrows, kch, xws_h, row_h, col_h, w_h, out_h,
               row_v, col_v, w_v, msg_v, zb_v, acc_sh, sem):
    c = lax.axis_index("c")
    s = lax.axis_index("s")
    wid = c * NS + s
    for k in range(nrows):
        zb_v[k, :] = jnp.zeros((L,), jnp.float32)
    pltpu.sync_copy(zb_v, acc_sh.at[pl.ds(s * nrows, nrows)])
    pltpu.sync_copy(row_h.at[wid], row_v)
    pltpu.sync_copy(col_h.at[wid], col_v)
    pltpu.sync_copy(w_h.at[wid], w_v)
    plsc.subcore_barrier()

    def body(j, carry):
        pltpu.async_copy(xws_h.at[row_v.at[j]], msg_v, sem).wait()
        base = j * CHUNK
        for ks in range(CHUNK // L):
            w16 = w_v[pl.ds(base + ks * L, L)]
            for t in range(L):
                e = ks * L + t
                wspl = w16.at[jnp.full((L,), t, jnp.int32)].get(
                    mode="promise_in_bounds")
                msg_v[e, :] = msg_v[e, :] * wspl
        pltpu.sync_copy(msg_v, acc_sh.at[col_v.at[j]], add=True)
        return carry

    lax.fori_loop(0, kch, body, 0)
    plsc.subcore_barrier()
    pltpu.sync_copy(acc_sh.at[pl.ds(s * nrows, nrows)],
                    out_h.at[c, pl.ds(s * nrows, nrows)])


def _make_deg_kernel(np_, kch):
    nrows = np_ // NS
    mesh = plsc.VectorSubcoreMesh(core_axis_name="c", subcore_axis_name="s")
    return pl.kernel(
        functools.partial(_deg_body, nrows, kch),
        out_type=jax.ShapeDtypeStruct((NC, np_), jnp.float32),
        mesh=mesh,
        scratch_types=[
            pltpu.VMEM((kch, CHUNK), jnp.int32),
            pltpu.VMEM((kch, CHUNK), jnp.float32),
            pltpu.VMEM((nrows,), jnp.float32),
            pltpu.VMEM_SHARED((np_,), jnp.float32),
        ],
    )


def _make_prop_kernel(np_, kch):
    nrows = np_ // NS
    mesh = plsc.VectorSubcoreMesh(core_axis_name="c", subcore_axis_name="s")
    return pl.kernel(
        functools.partial(_prop_body, nrows, kch),
        out_type=jax.ShapeDtypeStruct((NC, np_, L), jnp.float32),
        mesh=mesh,
        scratch_types=[
            pltpu.VMEM((kch, CHUNK), jnp.int32),
            pltpu.VMEM((kch, CHUNK), jnp.int32),
            pltpu.VMEM((kch * CHUNK,), jnp.float32),
            pltpu.VMEM((CHUNK, L), jnp.float32),
            pltpu.VMEM((nrows, L), jnp.float32),
            pltpu.VMEM_SHARED((np_, L), jnp.float32),
            pltpu.SemaphoreType.DMA,
        ],
        compiler_params=pltpu.CompilerParams(use_tc_tiling_on_sc=False),
    )


def _prep_body_tc(degp_ref, x_ref, w1_ref, xws_ref, dinvb_ref):
    deg = degp_ref[:, 0:1] + degp_ref[:, 1:2] + 1.0
    dinv = lax.rsqrt(deg)
    xw = jnp.dot(x_ref[...], w1_ref[...], preferred_element_type=jnp.float32)
    dinvb = jnp.broadcast_to(dinv, xw.shape)
    xws_ref[...] = xw * dinvb
    dinvb_ref[...] = dinvb


def _mid_body_tc(msgp_ref, xws1_ref, dinvb_ref, w2_ref, b1_ref, xws2_ref):
    agg = msgp_ref[0] + msgp_ref[1] + xws1_ref[...]
    h = jnp.maximum(dinvb_ref[...] * agg + b1_ref[...], 0.0)
    xw2 = jnp.dot(h, w2_ref[...], preferred_element_type=jnp.float32)
    xws2_ref[...] = xw2 * dinvb_ref[...]


def _out_body_tc(msgp_ref, xws2_ref, dinvb_ref, b2_ref, o_ref):
    o = dinvb_ref[...] * (msgp_ref[0] + msgp_ref[1] + xws2_ref[...]) + b2_ref[...]
    m = jnp.max(o, axis=1, keepdims=True)
    lse = jnp.log(jnp.sum(jnp.exp(o - m), axis=1, keepdims=True)) + m
    o_ref[...] = o - lse


def kernel(x, edge_index, edge_weight, W1, b1, W2, b2):
    n, f_in = x.shape
    e = edge_weight.shape[0]
    hid = W1.shape[1]
    out_f = W2.shape[1]
    assert hid == L and out_f == L

    nrows = _cdiv(_cdiv(n, NS), L) * L
    np_ = NS * nrows
    ept = _cdiv(e, NT)
    kch = _cdiv(ept, CHUNK)
    ep = NT * kch * CHUNK

    row = edge_index[0]
    col = edge_index[1]
    zpad_i = jnp.zeros((ep - e,), jnp.int32)
    rowp = jnp.concatenate([row, zpad_i]).reshape(NT, kch, CHUNK)
    colp = jnp.concatenate([col, zpad_i]).reshape(NT, kch, CHUNK)
    wp = jnp.concatenate([edge_weight, jnp.zeros((ep - e,), jnp.float32)])
    wp = wp.reshape(NT, kch, CHUNK)
    wp_flat = wp.reshape(NT, kch * CHUNK)
    x_p = jnp.pad(x, ((0, np_ - n), (0, 0)))
    b1r = b1.reshape(1, L)
    b2r = b2.reshape(1, L)

    degp = _make_deg_kernel(np_, kch)(colp, wp)
    degp_t = degp.T

    blk = 1024
    grid = (_cdiv(n, blk),)

    xws1, dinvb = pl.pallas_call(
        _prep_body_tc,
        grid=grid,
        in_specs=[
            pl.BlockSpec((blk, NC), lambda i: (i, 0)),
            pl.BlockSpec((blk, f_in), lambda i: (i, 0)),
            pl.BlockSpec((f_in, L), lambda i: (0, 0)),
        ],
        out_specs=[
            pl.BlockSpec((blk, L), lambda i: (i, 0)),
            pl.BlockSpec((blk, L), lambda i: (i, 0)),
        ],
        out_shape=[
            jax.ShapeDtypeStruct((np_, L), jnp.float32),
            jax.ShapeDtypeStruct((np_, L), jnp.float32),
        ],
    )(degp_t, x_p, W1)

    prop = _make_prop_kernel(np_, kch)

    msgp1 = prop(xws1, rowp, colp, wp_flat)

    xws2 = pl.pallas_call(
        _mid_body_tc,
        grid=grid,
        in_specs=[
            pl.BlockSpec((NC, blk, L), lambda i: (0, i, 0)),
            pl.BlockSpec((blk, L), lambda i: (i, 0)),
            pl.BlockSpec((blk, L), lambda i: (i, 0)),
            pl.BlockSpec((L, L), lambda i: (0, 0)),
            pl.BlockSpec((1, L), lambda i: (0, 0)),
        ],
        out_specs=pl.BlockSpec((blk, L), lambda i: (i, 0)),
        out_shape=jax.ShapeDtypeStruct((np_, L), jnp.float32),
    )(msgp1, xws1, dinvb, W2, b1r)

    msgp2 = prop(xws2, rowp, colp, wp_flat)

    out = pl.pallas_call(
        _out_body_tc,
        grid=grid,
        in_specs=[
            pl.BlockSpec((NC, blk, L), lambda i: (0, i, 0)),
            pl.BlockSpec((blk, L), lambda i: (i, 0)),
            pl.BlockSpec((blk, L), lambda i: (i, 0)),
            pl.BlockSpec((1, L), lambda i: (0, 0)),
        ],
        out_specs=pl.BlockSpec((blk, L), lambda i: (i, 0)),
        out_shape=jax.ShapeDtypeStruct((np_, L), jnp.float32),
    )(msgp2, xws2, dinvb, b2r)

    return out[:n]
